# baseline (speedup 1.0000x reference)
import functools

import jax
import jax.numpy as jnp
from jax import lax
from jax.experimental import pallas as pl
from jax.experimental.pallas import tpu as pltpu
from jax.experimental.pallas import tpu_sc as plsc

_N = 10000
_NP = 10112
_E = 320000
_D = 128
_K = 128
_NC, _NS = 2, 16
_NW = _NC * _NS
_CH = 80
_EP = _NW * _CH * _K
_NSEC = 2
_SCH = _CH // _NSEC
_RPT = _NP // _NS


@functools.partial(
    pl.kernel,
    mesh=plsc.VectorSubcoreMesh(core_axis_name="c", subcore_axis_name="s"),
    out_type=jax.ShapeDtypeStruct((_NC, _NP, _D), jnp.float32),
    scratch_types=[
        pltpu.VMEM((_SCH, _K), jnp.int32),
        pltpu.VMEM((_SCH, _K), jnp.int32),
        pltpu.VMEM((_K, _D), jnp.float32),
        pltpu.VMEM_SHARED((_NP, _D), jnp.float32),
        pltpu.SemaphoreType.DMA,
    ],
)
def _sc_segment_sum(src_hbm, dst_hbm, table_hbm, zeros_hbm, out_hbm,
                    sidx, didx, rows, acc, gsem):
    c = lax.axis_index("c")
    s = lax.axis_index("s")
    wid = s * _NC + c
    pltpu.sync_copy(zeros_hbm.at[pl.ds(s * _RPT, _RPT)],
                    acc.at[pl.ds(s * _RPT, _RPT)])
    plsc.subcore_barrier()

    def body(j, carry):
        pltpu.async_copy(table_hbm.at[sidx.at[j]], rows, gsem).wait()
        pltpu.sync_copy(rows, acc.at[didx.at[j]], add=True)
        return carry

    for sec in range(_NSEC):
        pltpu.sync_copy(src_hbm.at[wid, pl.ds(sec * _SCH, _SCH)], sidx)
        pltpu.sync_copy(dst_hbm.at[wid, pl.ds(sec * _SCH, _SCH)], didx)
        lax.fori_loop(0, _SCH, body, 0)

    plsc.subcore_barrier()
    pltpu.sync_copy(acc.at[pl.ds(s * _RPT, _RPT)],
                    out_hbm.at[c, pl.ds(s * _RPT, _RPT)])


def _combine_body(p_ref, o_ref):
    o_ref[...] = p_ref[0] + p_ref[1]


_ROWS_PER_BLK = 1000


def _combine(partials):
    return pl.pallas_call(
        _combine_body,
        out_shape=jax.ShapeDtypeStruct((_N, _D), jnp.float32),
        grid=(_N // _ROWS_PER_BLK,),
        in_specs=[pl.BlockSpec((_NC, _ROWS_PER_BLK, _D), lambda i: (0, i, 0))],
        out_specs=pl.BlockSpec((_ROWS_PER_BLK, _D), lambda i: (i, 0)),
    )(partials)


def kernel(source, target, edge_index, W_emb, b_emb, W_loc, b_loc, W_nb, b_nb):
    npad = _EP - _E
    pad_src = jnp.zeros((npad,), jnp.int32)
    pad_dst = _N + (jnp.arange(npad, dtype=jnp.int32) % (_NP - _N))
    src3d = jnp.concatenate([edge_index[0], pad_src]).reshape(_NW, _CH, _K)
    dst3d = jnp.concatenate([edge_index[1], pad_dst]).reshape(_NW, _CH, _K)
    zeros = jnp.zeros((_NP, _D), jnp.float32)
    partials = _sc_segment_sum(src3d, dst3d, source, zeros)
    return _combine(partials)


# speedup vs baseline: 2.8185x; 2.8185x over previous
import functools

import jax
import jax.numpy as jnp
from jax import lax
from jax.experimental import pallas as pl
from jax.experimental.pallas import tpu as pltpu
from jax.experimental.pallas import tpu_sc as plsc

_N = 10000
_NP = 10112
_E = 320000
_D = 128
_K = 80
_NC, _NS = 2, 16
_NW = _NC * _NS
_CH = 125
_SECS = (64, 61)
_RPT = _NP // _NS


@functools.partial(
    pl.kernel,
    mesh=plsc.VectorSubcoreMesh(core_axis_name="c", subcore_axis_name="s"),
    out_type=jax.ShapeDtypeStruct((_NC, _NP, _D), jnp.float32),
    scratch_types=[
        pltpu.VMEM((_SECS[0], _K), jnp.int32),
        pltpu.VMEM((_SECS[0], _K), jnp.int32),
        pltpu.VMEM((2, _K, _D), jnp.float32),
        pltpu.VMEM_SHARED((_NP, _D), jnp.float32),
        pltpu.SemaphoreType.DMA,
    ],
)
def _sc_segment_sum(src_hbm, dst_hbm, table_hbm, zeros_hbm, out_hbm,
                    sidx, didx, rows, acc, gsem):
    c = lax.axis_index("c")
    s = lax.axis_index("s")
    wid = s * _NC + c
    pltpu.sync_copy(zeros_hbm.at[pl.ds(s * _RPT, _RPT)],
                    acc.at[pl.ds(s * _RPT, _RPT)])
    plsc.subcore_barrier()

    def step(j, b, length):
        pltpu.make_async_copy(table_hbm.at[sidx.at[j]], rows.at[b], gsem).wait()

        @pl.when(j + 1 < length)
        def _():
            pltpu.async_copy(table_hbm.at[sidx.at[j + 1]], rows.at[1 - b], gsem)

        pltpu.sync_copy(rows.at[b], acc.at[didx.at[j]], add=True)

    off = 0
    for length in _SECS:
        pltpu.sync_copy(src_hbm.at[wid, pl.ds(off, length)],
                        sidx.at[pl.ds(0, length)])
        pltpu.sync_copy(dst_hbm.at[wid, pl.ds(off, length)],
                        didx.at[pl.ds(0, length)])
        pltpu.async_copy(table_hbm.at[sidx.at[0]], rows.at[0], gsem)

        def outer(i, carry, length=length):
            step(2 * i, 0, length)
            step(2 * i + 1, 1, length)
            return carry

        lax.fori_loop(0, length // 2, outer, 0)
        if length % 2:
            step(length - 1, 0, length)
        off += length

    plsc.subcore_barrier()
    pltpu.sync_copy(acc.at[pl.ds(s * _RPT, _RPT)],
                    out_hbm.at[c, pl.ds(s * _RPT, _RPT)])


def _combine_body(p_ref, o_ref):
    o_ref[...] = p_ref[0] + p_ref[1]


_ROWS_PER_BLK = 1000


def _combine(partials):
    return pl.pallas_call(
        _combine_body,
        out_shape=jax.ShapeDtypeStruct((_N, _D), jnp.float32),
        grid=(_N // _ROWS_PER_BLK,),
        in_specs=[pl.BlockSpec((_NC, _ROWS_PER_BLK, _D), lambda i: (0, i, 0))],
        out_specs=pl.BlockSpec((_ROWS_PER_BLK, _D), lambda i: (i, 0)),
    )(partials)


def kernel(source, target, edge_index, W_emb, b_emb, W_loc, b_loc, W_nb, b_nb):
    src3d = edge_index[0].reshape(_NW, _CH, _K)
    dst3d = edge_index[1].reshape(_NW, _CH, _K)
    zeros = jnp.zeros((_NP, _D), jnp.float32)
    partials = _sc_segment_sum(src3d, dst3d, source, zeros)
    return _combine(partials)
